# Initial kernel scaffold; baseline (speedup 1.0000x reference)
import jax
import jax.numpy as jnp
from jax.experimental import pallas as pl


def kernel(x, adj, W1, a1, W2, a2, Wl, bl):
    raise NotImplementedError("write your pallas kernel here")


# speedup vs baseline: 1.4896x; 1.4896x over previous
import functools

import jax
import jax.numpy as jnp
from jax.experimental import pallas as pl
from jax.experimental.pallas import tpu as pltpu

_N = 4096
_BR = 256


def _gat_block(h_ref, w_ref, af_ref, as_ref, adj_ref, out_ref,
               wh_s, e1_s, e2r_s, *, mode, wl_ref=None, bl_ref=None):
    i = pl.program_id(0)

    @pl.when(i == 0)
    def _init():
        wh = jnp.dot(h_ref[...], w_ref[...], preferred_element_type=jnp.float32)
        wh_s[...] = wh
        e1_s[...] = jnp.dot(wh, af_ref[...], preferred_element_type=jnp.float32)
        e2r_s[...] = jax.lax.dot_general(
            as_ref[...], wh, (((1,), (1,)), ((), ())),
            preferred_element_type=jnp.float32)

    adjb = adj_ref[...]
    e1b = e1_s[pl.ds(i * _BR, _BR), :]
    t = e1b + e2r_s[...]
    t = jnp.maximum(t, 0.2 * t)
    t = jnp.where(adjb > 0.0, t, jnp.float32(-9e15))
    m = jnp.max(t, axis=1, keepdims=True)
    p = jnp.exp(t - m)
    s = jnp.sum(p, axis=1, keepdims=True)
    o = jnp.dot(p, wh_s[...], preferred_element_type=jnp.float32) / s
    if mode == "elu":
        out_ref[...] = jnp.where(o > 0.0, o, jnp.exp(o) - 1.0)
    else:
        h2 = jax.nn.sigmoid(o)
        out_ref[...] = (
            jnp.dot(h2, wl_ref[...], preferred_element_type=jnp.float32)
            + bl_ref[...])


def _gat_layer1(x_pad, w_pad, a_first, a_snd, adj):
    grid = (_N // _BR,)
    return pl.pallas_call(
        functools.partial(_gat_block, mode="elu"),
        grid=grid,
        in_specs=[
            pl.BlockSpec((_N, 128), lambda i: (0, 0)),
            pl.BlockSpec((128, 128), lambda i: (0, 0)),
            pl.BlockSpec((128, 1), lambda i: (0, 0)),
            pl.BlockSpec((1, 128), lambda i: (0, 0)),
            pl.BlockSpec((_BR, _N), lambda i: (i, 0)),
        ],
        out_specs=pl.BlockSpec((_BR, 128), lambda i: (i, 0)),
        out_shape=jax.ShapeDtypeStruct((_N, 128), jnp.float32),
        scratch_shapes=[
            pltpu.VMEM((_N, 128), jnp.float32),
            pltpu.VMEM((_N, 1), jnp.float32),
            pltpu.VMEM((1, _N), jnp.float32),
        ],
    )(x_pad, w_pad, a_first, a_snd, adj)


def _gat_layer2(h, w2, a_first, a_snd, adj, wl_pad, bl_pad):
    grid = (_N // _BR,)

    def body(h_ref, w_ref, af_ref, as_ref, adj_ref, wl_ref, bl_ref, out_ref,
             wh_s, e1_s, e2r_s):
        _gat_block(h_ref, w_ref, af_ref, as_ref, adj_ref, out_ref,
                   wh_s, e1_s, e2r_s, mode="sig_lin",
                   wl_ref=wl_ref, bl_ref=bl_ref)

    return pl.pallas_call(
        body,
        grid=grid,
        in_specs=[
            pl.BlockSpec((_N, 128), lambda i: (0, 0)),
            pl.BlockSpec((128, 64), lambda i: (0, 0)),
            pl.BlockSpec((64, 1), lambda i: (0, 0)),
            pl.BlockSpec((1, 64), lambda i: (0, 0)),
            pl.BlockSpec((_BR, _N), lambda i: (i, 0)),
            pl.BlockSpec((64, 128), lambda i: (0, 0)),
            pl.BlockSpec((1, 128), lambda i: (0, 0)),
        ],
        out_specs=pl.BlockSpec((_BR, 128), lambda i: (i, 0)),
        out_shape=jax.ShapeDtypeStruct((_N, 128), jnp.float32),
        scratch_shapes=[
            pltpu.VMEM((_N, 64), jnp.float32),
            pltpu.VMEM((_N, 1), jnp.float32),
            pltpu.VMEM((1, _N), jnp.float32),
        ],
    )(h, w2, a_first, a_snd, adj, wl_pad, bl_pad)


def kernel(x, adj, W1, a1, W2, a2, Wl, bl):
    t = x.shape[1]
    x_pad = jnp.pad(x, ((0, 0), (0, 128 - t)))
    w1_pad = jnp.pad(W1, ((0, 128 - t), (0, 0)))
    h1 = _gat_layer1(x_pad, w1_pad,
                     a1[:128].reshape(128, 1), a1[128:].reshape(1, 128), adj)
    wl_pad = jnp.pad(Wl, ((0, 0), (0, 128 - Wl.shape[1])))
    bl_pad = jnp.pad(bl, (0, 128 - bl.shape[0])).reshape(1, 128)
    out = _gat_layer2(h1, W2,
                      a2[:64].reshape(64, 1), a2[64:].reshape(1, 64), adj,
                      wl_pad, bl_pad)
    return out[:, :Wl.shape[1]]


# speedup vs baseline: 1.7289x; 1.1607x over previous
import functools

import jax
import jax.numpy as jnp
from jax.experimental import pallas as pl
from jax.experimental.pallas import tpu as pltpu

_N = 4096
_BR = 256


def _gat_block(h_ref, w_ref, af_ref, as_ref, adj_ref, out_ref,
               whb_s, e1_s, e2r_s, e2m_s, *, mode, wl_ref=None, bl_ref=None):
    i = pl.program_id(0)

    @pl.when(i == 0)
    def _init():
        wh = jnp.dot(h_ref[...], w_ref[...], preferred_element_type=jnp.float32)
        whb_s[...] = wh.astype(jnp.bfloat16)
        e1_s[...] = jnp.dot(wh, af_ref[...], preferred_element_type=jnp.float32)
        e2r = jax.lax.dot_general(
            as_ref[...], wh, (((1,), (1,)), ((), ())),
            preferred_element_type=jnp.float32)
        e2r_s[...] = e2r
        e2m_s[...] = jnp.max(e2r, axis=1, keepdims=True)

    adjb = adj_ref[...]
    e1b = e1_s[pl.ds(i * _BR, _BR), :]
    tb = e1b + e2m_s[...]
    cb = jnp.maximum(tb, 0.2 * tb)
    t = e1b + e2r_s[...]
    u = jnp.maximum(t, 0.2 * t)
    p = jnp.exp(u - cb) * adjb
    s = jnp.sum(p, axis=1, keepdims=True)
    o = jnp.dot(p.astype(jnp.bfloat16), whb_s[...],
                preferred_element_type=jnp.float32) / s
    if mode == "elu":
        out_ref[...] = jnp.where(o > 0.0, o, jnp.exp(o) - 1.0)
    else:
        h2 = jax.nn.sigmoid(o)
        out_ref[...] = (
            jnp.dot(h2, wl_ref[...], preferred_element_type=jnp.float32)
            + bl_ref[...])


def _gat_layer1(x_pad, w_pad, a_first, a_snd, adj):
    grid = (_N // _BR,)
    return pl.pallas_call(
        functools.partial(_gat_block, mode="elu"),
        grid=grid,
        in_specs=[
            pl.BlockSpec((_N, 128), lambda i: (0, 0)),
            pl.BlockSpec((128, 128), lambda i: (0, 0)),
            pl.BlockSpec((128, 1), lambda i: (0, 0)),
            pl.BlockSpec((1, 128), lambda i: (0, 0)),
            pl.BlockSpec((_BR, _N), lambda i: (i, 0)),
        ],
        out_specs=pl.BlockSpec((_BR, 128), lambda i: (i, 0)),
        out_shape=jax.ShapeDtypeStruct((_N, 128), jnp.float32),
        scratch_shapes=[
            pltpu.VMEM((_N, 128), jnp.bfloat16),
            pltpu.VMEM((_N, 1), jnp.float32),
            pltpu.VMEM((1, _N), jnp.float32),
            pltpu.VMEM((1, 1), jnp.float32),
        ],
    )(x_pad, w_pad, a_first, a_snd, adj)


def _gat_layer2(h, w2, a_first, a_snd, adj, wl_pad, bl_pad):
    grid = (_N // _BR,)

    def body(h_ref, w_ref, af_ref, as_ref, adj_ref, wl_ref, bl_ref, out_ref,
             whb_s, e1_s, e2r_s, e2m_s):
        _gat_block(h_ref, w_ref, af_ref, as_ref, adj_ref, out_ref,
                   whb_s, e1_s, e2r_s, e2m_s, mode="sig_lin",
                   wl_ref=wl_ref, bl_ref=bl_ref)

    return pl.pallas_call(
        body,
        grid=grid,
        in_specs=[
            pl.BlockSpec((_N, 128), lambda i: (0, 0)),
            pl.BlockSpec((128, 64), lambda i: (0, 0)),
            pl.BlockSpec((64, 1), lambda i: (0, 0)),
            pl.BlockSpec((1, 64), lambda i: (0, 0)),
            pl.BlockSpec((_BR, _N), lambda i: (i, 0)),
            pl.BlockSpec((64, 128), lambda i: (0, 0)),
            pl.BlockSpec((1, 128), lambda i: (0, 0)),
        ],
        out_specs=pl.BlockSpec((_BR, 128), lambda i: (i, 0)),
        out_shape=jax.ShapeDtypeStruct((_N, 128), jnp.float32),
        scratch_shapes=[
            pltpu.VMEM((_N, 64), jnp.bfloat16),
            pltpu.VMEM((_N, 1), jnp.float32),
            pltpu.VMEM((1, _N), jnp.float32),
            pltpu.VMEM((1, 1), jnp.float32),
        ],
    )(h, w2, a_first, a_snd, adj, wl_pad, bl_pad)


def kernel(x, adj, W1, a1, W2, a2, Wl, bl):
    t = x.shape[1]
    x_pad = jnp.pad(x, ((0, 0), (0, 128 - t)))
    w1_pad = jnp.pad(W1, ((0, 128 - t), (0, 0)))
    h1 = _gat_layer1(x_pad, w1_pad,
                     a1[:128].reshape(128, 1), a1[128:].reshape(1, 128), adj)
    wl_pad = jnp.pad(Wl, ((0, 0), (0, 128 - Wl.shape[1])))
    bl_pad = jnp.pad(bl, (0, 128 - bl.shape[0])).reshape(1, 128)
    out = _gat_layer2(h1, W2,
                      a2[:64].reshape(64, 1), a2[64:].reshape(1, 64), adj,
                      wl_pad, bl_pad)
    return out[:, :Wl.shape[1]]


# speedup vs baseline: 1.9259x; 1.1139x over previous
import functools

import jax
import jax.numpy as jnp
from jax.experimental import pallas as pl
from jax.experimental.pallas import tpu as pltpu

_N = 4096
_BR = 512
_LOG2E = 1.4426950408889634


def _gat_block(h_ref, w_ref, af_ref, as_ref, adj_ref, out_ref,
               whb_s, ea_s, eb_s, ca_s, cb_s, *, mode,
               wl_ref=None, bl_ref=None):
    i = pl.program_id(0)

    @pl.when(i == 0)
    def _init():
        wh = jnp.dot(h_ref[...], w_ref[...], preferred_element_type=jnp.float32)
        whb_s[...] = wh.astype(jnp.bfloat16)
        e1 = jnp.dot(wh, af_ref[...], preferred_element_type=jnp.float32)
        e2r = jax.lax.dot_general(
            as_ref[...], wh, (((1,), (1,)), ((), ())),
            preferred_element_type=jnp.float32)
        e2m = jnp.max(e2r)
        tm = e1 + e2m
        c = jnp.maximum(tm, 0.2 * tm)
        ea_s[...] = (e1 - c) * _LOG2E
        eb_s[...] = (0.2 * e1 - c) * _LOG2E
        ca_s[...] = e2r * _LOG2E
        cb_s[...] = e2r * (0.2 * _LOG2E)

    adjb = adj_ref[...]
    rows = pl.ds(i * _BR, _BR)
    va = ea_s[rows, :] + ca_s[...]
    vb = eb_s[rows, :] + cb_s[...]
    p = jnp.exp2(jnp.maximum(va, vb)) * adjb
    s = jnp.sum(p, axis=1, keepdims=True)
    o = jnp.dot(p.astype(jnp.bfloat16), whb_s[...],
                preferred_element_type=jnp.float32) / s
    if mode == "elu":
        out_ref[...] = jnp.where(o > 0.0, o, jnp.exp(o) - 1.0)
    else:
        h2 = jax.nn.sigmoid(o)
        out_ref[...] = (
            jnp.dot(h2, wl_ref[...], preferred_element_type=jnp.float32)
            + bl_ref[...])


def _gat_layer1(x_pad, w_pad, a_first, a_snd, adj):
    grid = (_N // _BR,)
    return pl.pallas_call(
        functools.partial(_gat_block, mode="elu"),
        grid=grid,
        in_specs=[
            pl.BlockSpec((_N, 128), lambda i: (0, 0)),
            pl.BlockSpec((128, 128), lambda i: (0, 0)),
            pl.BlockSpec((128, 1), lambda i: (0, 0)),
            pl.BlockSpec((1, 128), lambda i: (0, 0)),
            pl.BlockSpec((_BR, _N), lambda i: (i, 0)),
        ],
        out_specs=pl.BlockSpec((_BR, 128), lambda i: (i, 0)),
        out_shape=jax.ShapeDtypeStruct((_N, 128), jnp.float32),
        scratch_shapes=[
            pltpu.VMEM((_N, 128), jnp.bfloat16),
            pltpu.VMEM((_N, 1), jnp.float32),
            pltpu.VMEM((_N, 1), jnp.float32),
            pltpu.VMEM((1, _N), jnp.float32),
            pltpu.VMEM((1, _N), jnp.float32),
        ],
    )(x_pad, w_pad, a_first, a_snd, adj)


def _gat_layer2(h, w2, a_first, a_snd, adj, wl_pad, bl_pad):
    grid = (_N // _BR,)

    def body(h_ref, w_ref, af_ref, as_ref, adj_ref, wl_ref, bl_ref, out_ref,
             whb_s, ea_s, eb_s, ca_s, cb_s):
        _gat_block(h_ref, w_ref, af_ref, as_ref, adj_ref, out_ref,
                   whb_s, ea_s, eb_s, ca_s, cb_s, mode="sig_lin",
                   wl_ref=wl_ref, bl_ref=bl_ref)

    return pl.pallas_call(
        body,
        grid=grid,
        in_specs=[
            pl.BlockSpec((_N, 128), lambda i: (0, 0)),
            pl.BlockSpec((128, 64), lambda i: (0, 0)),
            pl.BlockSpec((64, 1), lambda i: (0, 0)),
            pl.BlockSpec((1, 64), lambda i: (0, 0)),
            pl.BlockSpec((_BR, _N), lambda i: (i, 0)),
            pl.BlockSpec((64, 128), lambda i: (0, 0)),
            pl.BlockSpec((1, 128), lambda i: (0, 0)),
        ],
        out_specs=pl.BlockSpec((_BR, 128), lambda i: (i, 0)),
        out_shape=jax.ShapeDtypeStruct((_N, 128), jnp.float32),
        scratch_shapes=[
            pltpu.VMEM((_N, 64), jnp.bfloat16),
            pltpu.VMEM((_N, 1), jnp.float32),
            pltpu.VMEM((_N, 1), jnp.float32),
            pltpu.VMEM((1, _N), jnp.float32),
            pltpu.VMEM((1, _N), jnp.float32),
        ],
    )(h, w2, a_first, a_snd, adj, wl_pad, bl_pad)


def kernel(x, adj, W1, a1, W2, a2, Wl, bl):
    t = x.shape[1]
    x_pad = jnp.pad(x, ((0, 0), (0, 128 - t)))
    w1_pad = jnp.pad(W1, ((0, 128 - t), (0, 0)))
    h1 = _gat_layer1(x_pad, w1_pad,
                     a1[:128].reshape(128, 1), a1[128:].reshape(1, 128), adj)
    wl_pad = jnp.pad(Wl, ((0, 0), (0, 128 - Wl.shape[1])))
    bl_pad = jnp.pad(bl, (0, 128 - bl.shape[0])).reshape(1, 128)
    out = _gat_layer2(h1, W2,
                      a2[:64].reshape(64, 1), a2[64:].reshape(1, 64), adj,
                      wl_pad, bl_pad)
    return out[:, :Wl.shape[1]]


# speedup vs baseline: 2.2779x; 1.1828x over previous
import functools

import jax
import jax.numpy as jnp
from jax.experimental import pallas as pl
from jax.experimental.pallas import tpu as pltpu

_N = 4096
_BR = 512
_LOG2E = 1.4426950408889634


def _gat_block(h_ref, w_ref, af_ref, as_ref, adj_ref, out_ref,
               whb_s, ea_s, eb_s, ca_s, cb_s, *, mode,
               wl_ref=None, bl_ref=None):
    i = pl.program_id(0)

    @pl.when(i == 0)
    def _init():
        wh = jnp.dot(h_ref[...], w_ref[...], preferred_element_type=jnp.float32)
        f = w_ref.shape[1]
        whb = wh.astype(jnp.bfloat16)
        if f < 128:
            whb = jnp.concatenate(
                [whb, jnp.zeros((_N, 128 - f), jnp.bfloat16)], axis=1)
        whb_s[:, 0:128] = whb
        whb_s[:, 128:256] = jnp.ones((_N, 128), jnp.bfloat16)
        e1 = jnp.dot(wh, af_ref[...], preferred_element_type=jnp.float32)
        e2r = jax.lax.dot_general(
            as_ref[...], wh, (((1,), (1,)), ((), ())),
            preferred_element_type=jnp.float32)
        e2m = jnp.max(e2r)
        tm = e1 + e2m
        c = jnp.maximum(tm, 0.2 * tm)
        ea_s[...] = ((e1 - c) * _LOG2E).astype(jnp.bfloat16)
        eb_s[...] = ((0.2 * e1 - c) * _LOG2E).astype(jnp.bfloat16)
        ca_s[...] = (e2r * _LOG2E).astype(jnp.bfloat16)
        cb_s[...] = (e2r * (0.2 * _LOG2E)).astype(jnp.bfloat16)

    adjb = adj_ref[...].astype(jnp.bfloat16)
    rows = pl.ds(i * _BR, _BR)
    va = ea_s[rows, :] + ca_s[...]
    vb = eb_s[rows, :] + cb_s[...]
    p = jnp.exp2(jnp.maximum(va, vb)) * adjb
    o2 = jnp.dot(p, whb_s[...], preferred_element_type=jnp.float32)
    o = o2[:, 0:128] / o2[:, 128:256]
    if mode == "elu":
        out_ref[...] = jnp.where(o > 0.0, o, jnp.exp(o) - 1.0)
    else:
        h2 = jax.nn.sigmoid(o)
        out_ref[...] = (
            jnp.dot(h2, wl_ref[...], preferred_element_type=jnp.float32)
            + bl_ref[...])


def _gat_layer1(x_pad, w_pad, a_first, a_snd, adj):
    grid = (_N // _BR,)
    return pl.pallas_call(
        functools.partial(_gat_block, mode="elu"),
        grid=grid,
        in_specs=[
            pl.BlockSpec((_N, 128), lambda i: (0, 0)),
            pl.BlockSpec((128, 128), lambda i: (0, 0)),
            pl.BlockSpec((128, 1), lambda i: (0, 0)),
            pl.BlockSpec((1, 128), lambda i: (0, 0)),
            pl.BlockSpec((_BR, _N), lambda i: (i, 0)),
        ],
        out_specs=pl.BlockSpec((_BR, 128), lambda i: (i, 0)),
        out_shape=jax.ShapeDtypeStruct((_N, 128), jnp.float32),
        scratch_shapes=[
            pltpu.VMEM((_N, 256), jnp.bfloat16),
            pltpu.VMEM((_N, 1), jnp.bfloat16),
            pltpu.VMEM((_N, 1), jnp.bfloat16),
            pltpu.VMEM((1, _N), jnp.bfloat16),
            pltpu.VMEM((1, _N), jnp.bfloat16),
        ],
    )(x_pad, w_pad, a_first, a_snd, adj)


def _gat_layer2(h, w2, a_first, a_snd, adj, wl_pad, bl_pad):
    grid = (_N // _BR,)

    def body(h_ref, w_ref, af_ref, as_ref, adj_ref, wl_ref, bl_ref, out_ref,
             whb_s, ea_s, eb_s, ca_s, cb_s):
        _gat_block(h_ref, w_ref, af_ref, as_ref, adj_ref, out_ref,
                   whb_s, ea_s, eb_s, ca_s, cb_s, mode="sig_lin",
                   wl_ref=wl_ref, bl_ref=bl_ref)

    return pl.pallas_call(
        body,
        grid=grid,
        in_specs=[
            pl.BlockSpec((_N, 128), lambda i: (0, 0)),
            pl.BlockSpec((128, 64), lambda i: (0, 0)),
            pl.BlockSpec((64, 1), lambda i: (0, 0)),
            pl.BlockSpec((1, 64), lambda i: (0, 0)),
            pl.BlockSpec((_BR, _N), lambda i: (i, 0)),
            pl.BlockSpec((128, 128), lambda i: (0, 0)),
            pl.BlockSpec((1, 128), lambda i: (0, 0)),
        ],
        out_specs=pl.BlockSpec((_BR, 128), lambda i: (i, 0)),
        out_shape=jax.ShapeDtypeStruct((_N, 128), jnp.float32),
        scratch_shapes=[
            pltpu.VMEM((_N, 256), jnp.bfloat16),
            pltpu.VMEM((_N, 1), jnp.bfloat16),
            pltpu.VMEM((_N, 1), jnp.bfloat16),
            pltpu.VMEM((1, _N), jnp.bfloat16),
            pltpu.VMEM((1, _N), jnp.bfloat16),
        ],
    )(h, w2, a_first, a_snd, adj, wl_pad, bl_pad)


def kernel(x, adj, W1, a1, W2, a2, Wl, bl):
    t = x.shape[1]
    x_pad = jnp.pad(x, ((0, 0), (0, 128 - t)))
    w1_pad = jnp.pad(W1, ((0, 128 - t), (0, 0)))
    h1 = _gat_layer1(x_pad, w1_pad,
                     a1[:128].reshape(128, 1), a1[128:].reshape(1, 128), adj)
    wl_pad = jnp.pad(Wl, ((0, 128 - Wl.shape[0]), (0, 128 - Wl.shape[1])))
    bl_pad = jnp.pad(bl, (0, 128 - bl.shape[0])).reshape(1, 128)
    out = _gat_layer2(h1, W2,
                      a2[:64].reshape(64, 1), a2[64:].reshape(1, 64), adj,
                      wl_pad, bl_pad)
    return out[:, :Wl.shape[1]]
